# baseline (device time: 60472 ns/iter reference)
import jax
import jax.numpy as jnp
from jax import lax
from jax.experimental import pallas as pl
from jax.experimental.pallas import tpu as pltpu

N_DEV = 16
SQ = 1024
SKV = 1024
HQ_PER = 8
DH = 128
SCALE = 0.08838834764831843
PH = 256

BLOCK_ORDER = (0, 4, 8, 12, 1, 5, 9, 13, 2, 6, 10, 14, 3, 7, 11, 15)

A_RS_MASKS = (1, 4, 2, 8)
B_RS_MASKS = (4, 1, 8, 2)
RS_HALF = (256, 128, 64, 32)
A_ROFF = (0, 256, 384, 448)
B_ROFF = (480, 736, 864, 928)
A_AG_MASKS = (8, 2, 4, 1)
B_AG_MASKS = (2, 8, 1, 4)
AG_LEN = (32, 64, 128, 256)


def kernel(x, Wq, K_ext, V_ext, Wo):
    order = jnp.array(BLOCK_ORDER, dtype=jnp.int32)
    xb = (x[0].astype(jnp.bfloat16)
          .reshape(16, 64, SQ)[order].reshape(SQ, SQ))

    def body(x_ref, wq_ref, k_hbm, v_hbm, wo_ref, out_ref,
             k_t, v_t, wq_bf, wo_bf, ctx_buf, acc_ref, ag_buf, send_buf, recv_buf,
             k_sems, v_sems, rs_send_sems, rs_recv_sems,
             ag_send_sems, ag_recv_sems):
        my_pos = lax.axis_index("i")

        k_dmas, v_dmas = [], []
        for h in range(HQ_PER):
            kd = pltpu.make_async_copy(
                k_hbm.at[0, :, my_pos * HQ_PER + h, :], k_t.at[h], k_sems.at[h])
            vd = pltpu.make_async_copy(
                v_hbm.at[0, :, my_pos * HQ_PER + h, :], v_t.at[h], v_sems.at[h])
            kd.start()
            vd.start()
            k_dmas.append(kd)
            v_dmas.append(vd)

        wq_bf[:] = (wq_ref[:] * SCALE).astype(jnp.bfloat16)
        wo_bf[:] = wo_ref[:].astype(jnp.bfloat16)

        barrier_sem = pltpu.get_barrier_semaphore()
        for m in (1, 2, 4, 8):
            pl.semaphore_signal(barrier_sem, inc=1,
                                device_id=(jnp.bitwise_xor(my_pos, m),),
                                device_id_type=pl.DeviceIdType.MESH)
        pl.semaphore_wait(barrier_sem, 4)

        def gather_stripe(t_ref, h, g_idx):
            parts = [t_ref[h, pl.ds(64 * g_idx + 256 * j, 64), :]
                     for j in range(4)]
            return jnp.concatenate(parts, axis=0).astype(jnp.bfloat16)

        def compute_phase(g_idx, wait_dmas):
            goff = 256 * g_idx
            xg = x_ref[pl.ds(goff, PH), :]
            qg = jnp.dot(xg, wq_bf[:],
                         preferred_element_type=jnp.float32).astype(jnp.bfloat16)
            for h in range(HQ_PER):
                if wait_dmas:
                    k_dmas[h].wait()
                    v_dmas[h].wait()
                qh = qg[:, h * DH:(h + 1) * DH]
                kh = gather_stripe(k_t, h, g_idx)
                s = lax.dot_general(qh, kh, (((1,), (1,)), ((), ())),
                                    preferred_element_type=jnp.float32)
                p = jnp.exp(s)
                w = (p * (1.0 / jnp.sum(p, axis=1, keepdims=True))
                     ).astype(jnp.bfloat16)
                ctx_buf[:, h * DH:(h + 1) * DH] = jnp.dot(
                    w, gather_stripe(v_t, h, g_idx),
                    preferred_element_type=jnp.float32).astype(jnp.bfloat16)
            acc_ref[pl.ds(goff, PH), :] = jnp.dot(
                ctx_buf[:], wo_bf[:], preferred_element_type=jnp.float32)

        bitA = jnp.bitwise_and(my_pos, A_RS_MASKS[0]) != 0
        bitB = jnp.bitwise_and(my_pos, B_RS_MASKS[0]) != 0
        gA_send = jnp.where(bitA, 0, 1)
        gB_send = jnp.where(bitB, 2, 3)
        gA_keep = 1 - gA_send
        gB_keep = 5 - gB_send
        keepA = 256 * gA_keep
        keepB = 256 * gB_keep

        compute_phase(gA_send, wait_dmas=True)
        compute_phase(gB_send, wait_dmas=False)
        rdmas0 = []
        for st, (g_send, m) in enumerate(((gA_send, A_RS_MASKS[0]),
                                          (gB_send, B_RS_MASKS[0]))):
            sb = 256 * st
            send_buf[pl.ds(sb, PH), :] = (
                acc_ref[pl.ds(256 * g_send, PH), :].astype(jnp.bfloat16))
            rdma = pltpu.make_async_remote_copy(
                src_ref=send_buf.at[pl.ds(sb, PH), :],
                dst_ref=recv_buf.at[pl.ds((A_ROFF, B_ROFF)[st][0], PH), :],
                send_sem=rs_send_sems.at[st],
                recv_sem=rs_recv_sems.at[st],
                device_id=(jnp.bitwise_xor(my_pos, m),),
                device_id_type=pl.DeviceIdType.MESH,
            )
            rdma.start()
            rdmas0.append(rdma)
        offs = [keepA, keepB]
        roffs = (A_ROFF, B_ROFF)
        masks_rs = (A_RS_MASKS, B_RS_MASKS)
        masks_ag = (A_AG_MASKS, B_AG_MASKS)
        sboffs = (0, 256)
        pend = rdmas0

        def rs_acc(st, k):
            half = RS_HALF[k]
            pend[st].wait()
            acc_ref[pl.ds(offs[st], half), :] += (
                recv_buf[pl.ds(roffs[st][k], half), :].astype(jnp.float32))

        def rs_start(st, k):
            half = RS_HALF[k]
            m = masks_rs[st][k]
            partner = jnp.bitwise_xor(my_pos, m)
            bit = jnp.bitwise_and(my_pos, m) != 0
            send_off = offs[st] + jnp.where(bit, 0, half)
            offs[st] = offs[st] + jnp.where(bit, half, 0)
            send_buf[pl.ds(sboffs[st], half), :] = (
                acc_ref[pl.ds(send_off, half), :].astype(jnp.bfloat16))
            rdma = pltpu.make_async_remote_copy(
                src_ref=send_buf.at[pl.ds(sboffs[st], half), :],
                dst_ref=recv_buf.at[pl.ds(roffs[st][k], half), :],
                send_sem=rs_send_sems.at[2 * k + st],
                recv_sem=rs_recv_sems.at[2 * k + st],
                device_id=(partner,),
                device_id_type=pl.DeviceIdType.MESH,
            )
            rdma.start()
            pend[st] = rdma

        def ag_start(st, k):
            rdma = pltpu.make_async_remote_copy(
                src_ref=ag_buf.at[pl.ds(offs[st], AG_LEN[k]), :],
                dst_ref=ag_buf.at[pl.ds(offs[st], AG_LEN[k]), :],
                send_sem=ag_send_sems.at[2 * k + st],
                recv_sem=ag_recv_sems.at[2 * k + st],
                device_id=(jnp.bitwise_xor(my_pos, masks_ag[st][k]),),
                device_id_type=pl.DeviceIdType.MESH,
            )
            rdma.start()
            pend[st] = rdma

        def ag_fin(st, k):
            pend[st].wait()
            bit = jnp.bitwise_and(my_pos, masks_ag[st][k]) != 0
            offs[st] = offs[st] - jnp.where(bit, AG_LEN[k], 0)

        compute_phase(gA_keep, wait_dmas=False)
        rs_acc(0, 0)
        rs_start(0, 1)
        compute_phase(gB_keep, wait_dmas=False)
        rs_acc(1, 0)
        rs_start(1, 1)
        for k in range(2, 4):
            rs_acc(0, k - 1)
            rs_start(0, k)
            rs_acc(1, k - 1)
            rs_start(1, k)
        for st in range(2):
            rs_acc(st, 3)
            ag_buf[pl.ds(offs[st], 32), :] = (
                acc_ref[pl.ds(offs[st], 32), :].astype(jnp.bfloat16))
            ag_start(st, 0)
        for k in range(1, 4):
            ag_fin(0, k - 1)
            ag_start(0, k)
            ag_fin(1, k - 1)
            ag_start(1, k)
        ag_fin(0, 3)
        ag_fin(1, 3)

        for pb in range(16):
            out_ref[64 * BLOCK_ORDER[pb]:64 * BLOCK_ORDER[pb] + 64, :] = (
                ag_buf[64 * pb:64 * pb + 64, :])

    out = pl.pallas_call(
        body,
        out_shape=jax.ShapeDtypeStruct((SQ, SQ), jnp.bfloat16),
        in_specs=[
            pl.BlockSpec(memory_space=pltpu.VMEM),
            pl.BlockSpec(memory_space=pltpu.VMEM),
            pl.BlockSpec(memory_space=pl.ANY),
            pl.BlockSpec(memory_space=pl.ANY),
            pl.BlockSpec(memory_space=pltpu.VMEM),
        ],
        out_specs=pl.BlockSpec(memory_space=pltpu.VMEM),
        scratch_shapes=[
            pltpu.VMEM((HQ_PER, SKV, DH), jnp.float32),
            pltpu.VMEM((HQ_PER, SKV, DH), jnp.float32),
            pltpu.VMEM((SQ, SQ), jnp.bfloat16),
            pltpu.VMEM((SQ, SQ), jnp.bfloat16),
            pltpu.VMEM((PH, SQ), jnp.bfloat16),
            pltpu.VMEM((SQ, SQ), jnp.float32),
            pltpu.VMEM((SQ, SQ), jnp.bfloat16),
            pltpu.VMEM((512, SQ), jnp.bfloat16),
            pltpu.VMEM((960, SQ), jnp.bfloat16),
            pltpu.SemaphoreType.DMA((HQ_PER,)),
            pltpu.SemaphoreType.DMA((HQ_PER,)),
            pltpu.SemaphoreType.DMA((8,)),
            pltpu.SemaphoreType.DMA((8,)),
            pltpu.SemaphoreType.DMA((8,)),
            pltpu.SemaphoreType.DMA((8,)),
        ],
        compiler_params=pltpu.CompilerParams(collective_id=0),
    )(xb, Wq, K_ext, V_ext, Wo)
    return out[None, :, :]


# device time: 59377 ns/iter; 1.0184x vs baseline; 1.0184x over previous
import jax
import jax.numpy as jnp
from jax import lax
from jax.experimental import pallas as pl
from jax.experimental.pallas import tpu as pltpu

N_DEV = 16
SQ = 1024
SKV = 1024
HQ_PER = 8
DH = 128
SCALE = 0.08838834764831843
PH = 256

BLOCK_ORDER = (0, 4, 8, 12, 1, 5, 9, 13, 2, 6, 10, 14, 3, 7, 11, 15)

A_RS_MASKS = (1, 4, 2, 8)
B_RS_MASKS = (4, 1, 8, 2)
RS_HALF = (256, 128, 64, 32)
A_ROFF = (0, 256, 384, 448)
B_ROFF = (480, 736, 864, 928)
A_AG_MASKS = (8, 2, 4, 1)
B_AG_MASKS = (2, 8, 1, 4)
AG_LEN = (32, 64, 128, 256)


def kernel(x, Wq, K_ext, V_ext, Wo):
    order = jnp.array(BLOCK_ORDER, dtype=jnp.int32)
    xb = (x[0].astype(jnp.bfloat16)
          .reshape(16, 64, SQ)[order].reshape(SQ, SQ))
    Wqb = (Wq * SCALE).astype(jnp.bfloat16)
    Wob = Wo.astype(jnp.bfloat16)

    def body(x_ref, wq_ref, k_hbm, v_hbm, wo_ref, out_ref,
             k_t, v_t, ctx_buf, acc_ref, ag_buf, send_buf, recv_buf,
             k_sems, v_sems, rs_send_sems, rs_recv_sems,
             ag_send_sems, ag_recv_sems):
        my_pos = lax.axis_index("i")

        k_dmas, v_dmas = [], []
        for h in range(HQ_PER):
            kd = pltpu.make_async_copy(
                k_hbm.at[0, :, my_pos * HQ_PER + h, :], k_t.at[h], k_sems.at[h])
            vd = pltpu.make_async_copy(
                v_hbm.at[0, :, my_pos * HQ_PER + h, :], v_t.at[h], v_sems.at[h])
            kd.start()
            vd.start()
            k_dmas.append(kd)
            v_dmas.append(vd)

        barrier_sem = pltpu.get_barrier_semaphore()
        for m in (1, 2, 4, 8):
            pl.semaphore_signal(barrier_sem, inc=1,
                                device_id=(jnp.bitwise_xor(my_pos, m),),
                                device_id_type=pl.DeviceIdType.MESH)
        pl.semaphore_wait(barrier_sem, 4)

        def gather_stripe(t_ref, h, g_idx):
            parts = [t_ref[h, pl.ds(64 * g_idx + 256 * j, 64), :]
                     for j in range(4)]
            return jnp.concatenate(parts, axis=0).astype(jnp.bfloat16)

        def compute_phase(g_idx, wait_dmas):
            goff = 256 * g_idx
            xg = x_ref[pl.ds(goff, PH), :]
            qg = jnp.dot(xg, wq_ref[:],
                         preferred_element_type=jnp.float32).astype(jnp.bfloat16)
            for h in range(HQ_PER):
                if wait_dmas:
                    k_dmas[h].wait()
                    v_dmas[h].wait()
                qh = qg[:, h * DH:(h + 1) * DH]
                kh = gather_stripe(k_t, h, g_idx)
                s = lax.dot_general(qh, kh, (((1,), (1,)), ((), ())),
                                    preferred_element_type=jnp.float32)
                p = jnp.exp(s)
                w = (p * (1.0 / jnp.sum(p, axis=1, keepdims=True))
                     ).astype(jnp.bfloat16)
                ctx_buf[:, h * DH:(h + 1) * DH] = jnp.dot(
                    w, gather_stripe(v_t, h, g_idx),
                    preferred_element_type=jnp.float32).astype(jnp.bfloat16)
            acc_ref[pl.ds(goff, PH), :] = jnp.dot(
                ctx_buf[:], wo_ref[:], preferred_element_type=jnp.float32)

        bitA = jnp.bitwise_and(my_pos, A_RS_MASKS[0]) != 0
        bitB = jnp.bitwise_and(my_pos, B_RS_MASKS[0]) != 0
        gA_send = jnp.where(bitA, 0, 1)
        gB_send = jnp.where(bitB, 2, 3)
        gA_keep = 1 - gA_send
        gB_keep = 5 - gB_send
        keepA = 256 * gA_keep
        keepB = 256 * gB_keep

        compute_phase(gA_send, wait_dmas=True)
        compute_phase(gB_send, wait_dmas=False)
        rdmas0 = []
        for st, (g_send, m) in enumerate(((gA_send, A_RS_MASKS[0]),
                                          (gB_send, B_RS_MASKS[0]))):
            sb = 256 * st
            send_buf[pl.ds(sb, PH), :] = (
                acc_ref[pl.ds(256 * g_send, PH), :].astype(jnp.bfloat16))
            rdma = pltpu.make_async_remote_copy(
                src_ref=send_buf.at[pl.ds(sb, PH), :],
                dst_ref=recv_buf.at[pl.ds((A_ROFF, B_ROFF)[st][0], PH), :],
                send_sem=rs_send_sems.at[st],
                recv_sem=rs_recv_sems.at[st],
                device_id=(jnp.bitwise_xor(my_pos, m),),
                device_id_type=pl.DeviceIdType.MESH,
            )
            rdma.start()
            rdmas0.append(rdma)
        compute_phase(gA_keep, wait_dmas=False)
        compute_phase(gB_keep, wait_dmas=False)

        offs = [keepA, keepB]
        roffs = (A_ROFF, B_ROFF)
        masks_rs = (A_RS_MASKS, B_RS_MASKS)
        masks_ag = (A_AG_MASKS, B_AG_MASKS)
        sboffs = (0, 256)
        pend = rdmas0

        def rs_acc(st, k):
            half = RS_HALF[k]
            pend[st].wait()
            acc_ref[pl.ds(offs[st], half), :] += (
                recv_buf[pl.ds(roffs[st][k], half), :].astype(jnp.float32))

        def rs_start(st, k):
            half = RS_HALF[k]
            m = masks_rs[st][k]
            partner = jnp.bitwise_xor(my_pos, m)
            bit = jnp.bitwise_and(my_pos, m) != 0
            send_off = offs[st] + jnp.where(bit, 0, half)
            offs[st] = offs[st] + jnp.where(bit, half, 0)
            send_buf[pl.ds(sboffs[st], half), :] = (
                acc_ref[pl.ds(send_off, half), :].astype(jnp.bfloat16))
            rdma = pltpu.make_async_remote_copy(
                src_ref=send_buf.at[pl.ds(sboffs[st], half), :],
                dst_ref=recv_buf.at[pl.ds(roffs[st][k], half), :],
                send_sem=rs_send_sems.at[2 * k + st],
                recv_sem=rs_recv_sems.at[2 * k + st],
                device_id=(partner,),
                device_id_type=pl.DeviceIdType.MESH,
            )
            rdma.start()
            pend[st] = rdma

        def ag_start(st, k):
            rdma = pltpu.make_async_remote_copy(
                src_ref=ag_buf.at[pl.ds(offs[st], AG_LEN[k]), :],
                dst_ref=ag_buf.at[pl.ds(offs[st], AG_LEN[k]), :],
                send_sem=ag_send_sems.at[2 * k + st],
                recv_sem=ag_recv_sems.at[2 * k + st],
                device_id=(jnp.bitwise_xor(my_pos, masks_ag[st][k]),),
                device_id_type=pl.DeviceIdType.MESH,
            )
            rdma.start()
            pend[st] = rdma

        def ag_fin(st, k):
            pend[st].wait()
            bit = jnp.bitwise_and(my_pos, masks_ag[st][k]) != 0
            offs[st] = offs[st] - jnp.where(bit, AG_LEN[k], 0)

        for k in range(1, 4):
            rs_acc(0, k - 1)
            rs_start(0, k)
            rs_acc(1, k - 1)
            rs_start(1, k)
        for st in range(2):
            rs_acc(st, 3)
            ag_buf[pl.ds(offs[st], 32), :] = (
                acc_ref[pl.ds(offs[st], 32), :].astype(jnp.bfloat16))
            ag_start(st, 0)
        for k in range(1, 4):
            ag_fin(0, k - 1)
            ag_start(0, k)
            ag_fin(1, k - 1)
            ag_start(1, k)
        ag_fin(0, 3)
        ag_fin(1, 3)

        for pb in range(16):
            out_ref[64 * BLOCK_ORDER[pb]:64 * BLOCK_ORDER[pb] + 64, :] = (
                ag_buf[64 * pb:64 * pb + 64, :])

    out = pl.pallas_call(
        body,
        out_shape=jax.ShapeDtypeStruct((SQ, SQ), jnp.bfloat16),
        in_specs=[
            pl.BlockSpec(memory_space=pltpu.VMEM),
            pl.BlockSpec(memory_space=pltpu.VMEM),
            pl.BlockSpec(memory_space=pl.ANY),
            pl.BlockSpec(memory_space=pl.ANY),
            pl.BlockSpec(memory_space=pltpu.VMEM),
        ],
        out_specs=pl.BlockSpec(memory_space=pltpu.VMEM),
        scratch_shapes=[
            pltpu.VMEM((HQ_PER, SKV, DH), jnp.float32),
            pltpu.VMEM((HQ_PER, SKV, DH), jnp.float32),
            pltpu.VMEM((PH, SQ), jnp.bfloat16),
            pltpu.VMEM((SQ, SQ), jnp.float32),
            pltpu.VMEM((SQ, SQ), jnp.bfloat16),
            pltpu.VMEM((512, SQ), jnp.bfloat16),
            pltpu.VMEM((960, SQ), jnp.bfloat16),
            pltpu.SemaphoreType.DMA((HQ_PER,)),
            pltpu.SemaphoreType.DMA((HQ_PER,)),
            pltpu.SemaphoreType.DMA((8,)),
            pltpu.SemaphoreType.DMA((8,)),
            pltpu.SemaphoreType.DMA((8,)),
            pltpu.SemaphoreType.DMA((8,)),
        ],
        compiler_params=pltpu.CompilerParams(collective_id=0),
    )(xb, Wqb, K_ext, V_ext, Wob)
    return out[None, :, :]
